# Initial kernel scaffold; baseline (speedup 1.0000x reference)
import jax
import jax.numpy as jnp
from jax.experimental import pallas as pl


def kernel(hosps, loc_ind, offset, alert, baseline_features, eff_features, index, spatial_features, Wb1, bb1, Wb2, bb2, We1, be1, We2, be2):
    raise NotImplementedError("write your pallas kernel here")


# speedup vs baseline: 59.1765x; 59.1765x over previous
import functools

import jax
import jax.numpy as jnp
from jax import lax
from jax.experimental import pallas as pl
from jax.experimental.pallas import tpu as pltpu
from jax.experimental.pallas import tpu_sc as plsc

S = 3000
SP = 3072
D_BASE = 8
D_EFF = 4
D_TAB = D_BASE + D_EFF
N = 1048576

NC = 2
NS = 16
NW = NC * NS
PER_W = N // NW
CHUNK = 2048
GROUPS = CHUNK // 16


def _table_tc_body(xt_ref, w1t_ref, b1_ref, w2t_ref, b2_ref, out_ref):
    h = jnp.dot(w1t_ref[...], xt_ref[...], preferred_element_type=jnp.float32)
    h = h + b1_ref[...]
    h = h * (1.0 / (1.0 + jnp.exp(-h)))
    t = jnp.dot(w2t_ref[...], h, preferred_element_type=jnp.float32)
    t = t + b2_ref[...]
    r = lax.broadcasted_iota(jnp.int32, (16, SP), 0)
    e = jnp.exp(t)
    t = jnp.where((r == 0) | (r == 8), e, jnp.where(r == 1, -e, t))
    out_ref[...] = t


def _make_table(spatial_features, Wb1, bb1, Wb2, bb2, We1, be1, We2, be2):
    xt = jnp.zeros((32, SP), jnp.float32).at[:, :S].set(spatial_features.T)
    w1t = jnp.concatenate([Wb1.T, We1.T], axis=0)
    b1 = jnp.concatenate([bb1, be1])[:, None]
    w2t = jnp.zeros((16, 64), jnp.float32)
    w2t = w2t.at[:D_BASE, :32].set(Wb2.T).at[D_BASE:D_TAB, 32:].set(We2.T)
    b2 = jnp.zeros((16,), jnp.float32)
    b2 = b2.at[:D_BASE].set(bb2).at[D_BASE:D_TAB].set(be2)
    return pl.pallas_call(
        _table_tc_body,
        out_shape=jax.ShapeDtypeStruct((16, SP), jnp.float32),
    )(xt, w1t, b1, w2t, b2[:, None])


def _sc_body(tt_hbm, loc_hbm, off_hbm, al_hbm, bf_hbm, ef_hbm, out_hbm,
             tab_v, loc_v, off_v, al_v, bf_v, ef_v, out_v):
    wid = lax.axis_index("c") * NS + lax.axis_index("s")
    w0 = wid * PER_W
    pltpu.sync_copy(tt_hbm, tab_v)

    lanes = lax.broadcasted_iota(jnp.int32, (16,), 0)

    def chunk_body(ci, _):
        row0 = w0 + ci * CHUNK
        pltpu.sync_copy(loc_hbm.at[pl.ds(row0, CHUNK)], loc_v)
        pltpu.sync_copy(off_hbm.at[pl.ds(row0, CHUNK)], off_v)
        pltpu.sync_copy(al_hbm.at[pl.ds(row0, CHUNK)], al_v)
        pltpu.sync_copy(bf_hbm.at[pl.ds(row0, CHUNK)], bf_v)
        pltpu.sync_copy(ef_hbm.at[pl.ds(row0, CHUNK)], ef_v)

        def group_body(g, _):
            base = g * 16
            rows = base + lanes
            idx = loc_v[pl.ds(base, 16)]
            acc_b = jnp.zeros((16,), jnp.float32)
            for j in range(D_BASE):
                jv = jnp.full((16,), j, jnp.int32)
                cj = plsc.load_gather(tab_v, [jv, idx])
                fj = plsc.load_gather(bf_v, [rows, jv])
                acc_b = acc_b + cj * fj
            acc_e = jnp.zeros((16,), jnp.float32)
            for j in range(D_EFF):
                jv = jnp.full((16,), j, jnp.int32)
                cj = plsc.load_gather(tab_v, [jnp.full((16,), D_BASE + j, jnp.int32), idx])
                fj = plsc.load_gather(ef_v, [rows, jv])
                acc_e = acc_e + cj * fj
            baseline = jnp.exp(jnp.minimum(acc_b, 10.0))
            eff = 1.0 / (1.0 + jnp.exp(4.0 - acc_e))
            eff = jnp.clip(eff, 1e-6, 1.0 - 1e-6)
            ov = off_v[pl.ds(base, 16)]
            av = al_v[pl.ds(base, 16)]
            outcome = ov * baseline * (1.0 - av * eff)
            plsc.store_scatter(out_v, [rows, jnp.zeros((16,), jnp.int32)], eff)
            plsc.store_scatter(out_v, [rows, jnp.full((16,), 1, jnp.int32)], baseline)
            plsc.store_scatter(out_v, [rows, jnp.full((16,), 2, jnp.int32)], outcome)
            return ()

        lax.fori_loop(0, GROUPS, group_body, (), unroll=False)
        pltpu.sync_copy(out_v, out_hbm.at[pl.ds(row0, CHUNK)])
        return ()

    lax.fori_loop(0, PER_W // CHUNK, chunk_body, (), unroll=False)


def _sc_compute(tt, loc_ind, offset, alert, baseline_features, eff_features):
    mesh = plsc.VectorSubcoreMesh(core_axis_name="c", subcore_axis_name="s")
    f = pl.kernel(
        _sc_body,
        out_type=jax.ShapeDtypeStruct((N, 3), jnp.float32),
        mesh=mesh,
        compiler_params=pltpu.CompilerParams(
            needs_layout_passes=False, use_tc_tiling_on_sc=False),
        scratch_types=[
            pltpu.VMEM((16, SP), jnp.float32),
            pltpu.VMEM((CHUNK,), jnp.int32),
            pltpu.VMEM((CHUNK,), jnp.float32),
            pltpu.VMEM((CHUNK,), jnp.float32),
            pltpu.VMEM((CHUNK, D_BASE), jnp.float32),
            pltpu.VMEM((CHUNK, D_EFF), jnp.float32),
            pltpu.VMEM((CHUNK, 3), jnp.float32),
        ],
    )
    return f(tt, loc_ind, offset, alert, baseline_features, eff_features)


def kernel(hosps, loc_ind, offset, alert, baseline_features, eff_features, index,
           spatial_features, Wb1, bb1, Wb2, bb2, We1, be1, We2, be2):
    tt = _make_table(spatial_features, Wb1, bb1, Wb2, bb2, We1, be1, We2, be2)
    return _sc_compute(tt, loc_ind, offset, alert, baseline_features, eff_features)


# speedup vs baseline: 61.3963x; 1.0375x over previous
import jax
import jax.numpy as jnp
from jax import lax
from jax.experimental import pallas as pl
from jax.experimental.pallas import tpu as pltpu
from jax.experimental.pallas import tpu_sc as plsc

S = 3000
SP = 3072
D_BASE = 8
D_EFF = 4
D_TAB = D_BASE + D_EFF
N = 1048576

NC = 2
NS = 16
NW = NC * NS
PER_W = N // NW
CHUNK = 2048
GROUPS = CHUNK // 16


def _table_tc_body(xt_ref, w1t_ref, b1_ref, w2t_ref, b2_ref, out_ref):
    h = jnp.dot(w1t_ref[...], xt_ref[...], preferred_element_type=jnp.float32)
    h = h + b1_ref[...]
    h = h * (1.0 / (1.0 + jnp.exp(-h)))
    t = jnp.dot(w2t_ref[...], h, preferred_element_type=jnp.float32)
    t = t + b2_ref[...]
    r = lax.broadcasted_iota(jnp.int32, (16, SP), 0)
    e = jnp.exp(t)
    t = jnp.where((r == 0) | (r == 8), e, jnp.where(r == 1, -e, t))
    out_ref[...] = t


def _make_table(spatial_features, Wb1, bb1, Wb2, bb2, We1, be1, We2, be2):
    xt = jnp.zeros((32, SP), jnp.float32).at[:, :S].set(spatial_features.T)
    w1t = jnp.concatenate([Wb1.T, We1.T], axis=0)
    b1 = jnp.concatenate([bb1, be1])[:, None]
    w2t = jnp.zeros((16, 64), jnp.float32)
    w2t = w2t.at[:D_BASE, :32].set(Wb2.T).at[D_BASE:D_TAB, 32:].set(We2.T)
    b2 = jnp.zeros((16,), jnp.float32)
    b2 = b2.at[:D_BASE].set(bb2).at[D_BASE:D_TAB].set(be2)
    return pl.pallas_call(
        _table_tc_body,
        out_shape=jax.ShapeDtypeStruct((16, SP), jnp.float32),
    )(xt, w1t, b1, w2t, b2[:, None])


def _sc_body(tt_hbm, loc_hbm, off_hbm, al_hbm, bf_hbm, ef_hbm, out_hbm,
             tab_v, loc_v, off_v, al_v, bf_v, ef_v, out_v):
    wid = lax.axis_index("c") * NS + lax.axis_index("s")
    w0 = wid * PER_W
    pltpu.sync_copy(tt_hbm, tab_v)

    lanes = lax.broadcasted_iota(jnp.int32, (16,), 0)

    def chunk_body(ci, _):
        row0 = w0 + ci * CHUNK
        pltpu.sync_copy(loc_hbm.at[pl.ds(row0, CHUNK)], loc_v)
        pltpu.sync_copy(off_hbm.at[pl.ds(row0, CHUNK)], off_v)
        pltpu.sync_copy(al_hbm.at[pl.ds(row0, CHUNK)], al_v)
        pltpu.sync_copy(bf_hbm.at[pl.ds(row0 * D_BASE, CHUNK * D_BASE)], bf_v)
        pltpu.sync_copy(ef_hbm.at[pl.ds(row0 * D_EFF, CHUNK * D_EFF)], ef_v)

        def group_body(g, _):
            base = g * 16
            idx = loc_v[pl.ds(base, 16)]
            rows8 = (base * D_BASE) + lanes * D_BASE
            rows4 = (base * D_EFF) + lanes * D_EFF
            acc_b = jnp.zeros((16,), jnp.float32)
            for j in range(D_BASE):
                cj = plsc.load_gather(tab_v, [idx + (j * SP)])
                fj = plsc.load_gather(bf_v, [rows8 + j])
                acc_b = acc_b + cj * fj
            acc_e = jnp.zeros((16,), jnp.float32)
            for j in range(D_EFF):
                cj = plsc.load_gather(tab_v, [idx + ((D_BASE + j) * SP)])
                fj = plsc.load_gather(ef_v, [rows4 + j])
                acc_e = acc_e + cj * fj
            baseline = jnp.exp(jnp.minimum(acc_b, 10.0))
            eff = 1.0 / (1.0 + jnp.exp(4.0 - acc_e))
            eff = jnp.clip(eff, 1e-6, 1.0 - 1e-6)
            ov = off_v[pl.ds(base, 16)]
            av = al_v[pl.ds(base, 16)]
            outcome = ov * baseline * (1.0 - av * eff)
            rows3 = (base * 3) + lanes * 3
            plsc.store_scatter(out_v, [rows3], eff)
            plsc.store_scatter(out_v, [rows3 + 1], baseline)
            plsc.store_scatter(out_v, [rows3 + 2], outcome)
            return ()

        lax.fori_loop(0, GROUPS, group_body, (), unroll=False)
        pltpu.sync_copy(out_v, out_hbm.at[pl.ds(row0 * 3, CHUNK * 3)])
        return ()

    lax.fori_loop(0, PER_W // CHUNK, chunk_body, (), unroll=False)


def _sc_compute(tt, loc_ind, offset, alert, bf_flat, ef_flat):
    mesh = plsc.VectorSubcoreMesh(core_axis_name="c", subcore_axis_name="s")
    f = pl.kernel(
        _sc_body,
        out_type=jax.ShapeDtypeStruct((N * 3,), jnp.float32),
        mesh=mesh,
        compiler_params=pltpu.CompilerParams(
            needs_layout_passes=False, use_tc_tiling_on_sc=False),
        scratch_types=[
            pltpu.VMEM((16 * SP,), jnp.float32),
            pltpu.VMEM((CHUNK,), jnp.int32),
            pltpu.VMEM((CHUNK,), jnp.float32),
            pltpu.VMEM((CHUNK,), jnp.float32),
            pltpu.VMEM((CHUNK * D_BASE,), jnp.float32),
            pltpu.VMEM((CHUNK * D_EFF,), jnp.float32),
            pltpu.VMEM((CHUNK * 3,), jnp.float32),
        ],
    )
    return f(tt, loc_ind, offset, alert, bf_flat, ef_flat)


def kernel(hosps, loc_ind, offset, alert, baseline_features, eff_features, index,
           spatial_features, Wb1, bb1, Wb2, bb2, We1, be1, We2, be2):
    tt = _make_table(spatial_features, Wb1, bb1, Wb2, bb2, We1, be1, We2, be2)
    out_flat = _sc_compute(tt.reshape(-1), loc_ind, offset, alert,
                           baseline_features.reshape(-1),
                           eff_features.reshape(-1))
    return out_flat.reshape(N, 3)


# speedup vs baseline: 755.3115x; 12.3022x over previous
import jax
import jax.numpy as jnp
from jax import lax
from jax.experimental import pallas as pl
from jax.experimental.pallas import tpu as pltpu
from jax.experimental.pallas import tpu_sc as plsc

S = 3000
SP = 3072
D_BASE = 8
D_EFF = 4
D_TAB = D_BASE + D_EFF
N = 1048576

NC = 2
NS = 16
NW = NC * NS
PER_W = N // NW
CHUNK = 2048
GROUPS = CHUNK // 16


def _table_tc_body(xt_ref, w1t_ref, b1_ref, w2t_ref, b2_ref, out_ref):
    h = jnp.dot(w1t_ref[...], xt_ref[...], preferred_element_type=jnp.float32)
    h = h + b1_ref[...]
    h = h * (1.0 / (1.0 + jnp.exp(-h)))
    t = jnp.dot(w2t_ref[...], h, preferred_element_type=jnp.float32)
    t = t + b2_ref[...]
    r = lax.broadcasted_iota(jnp.int32, (16, SP), 0)
    e = jnp.exp(t)
    t = jnp.where((r == 0) | (r == 8), e, jnp.where(r == 1, -e, t))
    out_ref[...] = t


def _make_table(spatial_features, Wb1, bb1, Wb2, bb2, We1, be1, We2, be2):
    xt = jnp.zeros((32, SP), jnp.float32).at[:, :S].set(spatial_features.T)
    w1t = jnp.concatenate([Wb1.T, We1.T], axis=0)
    b1 = jnp.concatenate([bb1, be1])[:, None]
    w2t = jnp.zeros((16, 64), jnp.float32)
    w2t = w2t.at[:D_BASE, :32].set(Wb2.T).at[D_BASE:D_TAB, 32:].set(We2.T)
    b2 = jnp.zeros((16,), jnp.float32)
    b2 = b2.at[:D_BASE].set(bb2).at[D_BASE:D_TAB].set(be2)
    return pl.pallas_call(
        _table_tc_body,
        out_shape=jax.ShapeDtypeStruct((16, SP), jnp.float32),
    )(xt, w1t, b1, w2t, b2[:, None])


def _sc_body(tt_hbm, loc_hbm, off_hbm, al_hbm, bft_hbm, eft_hbm,
             eff_hbm, base_hbm, outc_hbm,
             tab_v, loc_v, off_v, al_v, bf_v, ef_v, eff_o, base_o, outc_o):
    wid = lax.axis_index("c") * NS + lax.axis_index("s")
    w0 = wid * PER_W
    pltpu.sync_copy(tt_hbm, tab_v)

    lanes = lax.broadcasted_iota(jnp.int32, (16,), 0)

    def chunk_body(ci, _):
        row0 = w0 + ci * CHUNK
        pltpu.sync_copy(loc_hbm.at[pl.ds(row0, CHUNK)], loc_v)
        pltpu.sync_copy(off_hbm.at[pl.ds(row0, CHUNK)], off_v)
        pltpu.sync_copy(al_hbm.at[pl.ds(row0, CHUNK)], al_v)
        pltpu.sync_copy(bft_hbm.at[:, pl.ds(row0, CHUNK)], bf_v)
        pltpu.sync_copy(eft_hbm.at[:, pl.ds(row0, CHUNK)], ef_v)

        def group_body(g, _):
            base = g * 16
            idx = loc_v[pl.ds(base, 16)]
            acc_b = jnp.zeros((16,), jnp.float32)
            for j in range(D_BASE):
                cj = plsc.load_gather(tab_v, [idx + (j * SP)])
                fj = bf_v[j, pl.ds(base, 16)]
                acc_b = acc_b + cj * fj
            acc_e = jnp.zeros((16,), jnp.float32)
            for j in range(D_EFF):
                cj = plsc.load_gather(tab_v, [idx + ((D_BASE + j) * SP)])
                fj = ef_v[j, pl.ds(base, 16)]
                acc_e = acc_e + cj * fj
            baseline = jnp.exp(jnp.minimum(acc_b, 10.0))
            eff = 1.0 / (1.0 + jnp.exp(4.0 - acc_e))
            eff = jnp.clip(eff, 1e-6, 1.0 - 1e-6)
            ov = off_v[pl.ds(base, 16)]
            av = al_v[pl.ds(base, 16)]
            outcome = ov * baseline * (1.0 - av * eff)
            eff_o[pl.ds(base, 16)] = eff
            base_o[pl.ds(base, 16)] = baseline
            outc_o[pl.ds(base, 16)] = outcome
            return ()

        lax.fori_loop(0, GROUPS, group_body, (), unroll=False)
        pltpu.sync_copy(eff_o, eff_hbm.at[pl.ds(row0, CHUNK)])
        pltpu.sync_copy(base_o, base_hbm.at[pl.ds(row0, CHUNK)])
        pltpu.sync_copy(outc_o, outc_hbm.at[pl.ds(row0, CHUNK)])
        return ()

    lax.fori_loop(0, PER_W // CHUNK, chunk_body, (), unroll=False)


def _sc_compute(tt_flat, loc_ind, offset, alert, bf_t, ef_t):
    mesh = plsc.VectorSubcoreMesh(core_axis_name="c", subcore_axis_name="s")
    f = pl.kernel(
        _sc_body,
        out_type=(jax.ShapeDtypeStruct((N,), jnp.float32),
                  jax.ShapeDtypeStruct((N,), jnp.float32),
                  jax.ShapeDtypeStruct((N,), jnp.float32)),
        mesh=mesh,
        compiler_params=pltpu.CompilerParams(
            needs_layout_passes=False, use_tc_tiling_on_sc=True),
        scratch_types=[
            pltpu.VMEM((16 * SP,), jnp.float32),
            pltpu.VMEM((CHUNK,), jnp.int32),
            pltpu.VMEM((CHUNK,), jnp.float32),
            pltpu.VMEM((CHUNK,), jnp.float32),
            pltpu.VMEM((D_BASE, CHUNK), jnp.float32),
            pltpu.VMEM((D_EFF, CHUNK), jnp.float32),
            pltpu.VMEM((CHUNK,), jnp.float32),
            pltpu.VMEM((CHUNK,), jnp.float32),
            pltpu.VMEM((CHUNK,), jnp.float32),
        ],
    )
    return f(tt_flat, loc_ind, offset, alert, bf_t, ef_t)


def kernel(hosps, loc_ind, offset, alert, baseline_features, eff_features, index,
           spatial_features, Wb1, bb1, Wb2, bb2, We1, be1, We2, be2):
    tt = _make_table(spatial_features, Wb1, bb1, Wb2, bb2, We1, be1, We2, be2)
    eff, baseline, outcome = _sc_compute(
        tt.reshape(-1), loc_ind, offset, alert,
        baseline_features.T, eff_features.T)
    return jnp.stack([eff, baseline, outcome], axis=1)


# speedup vs baseline: 1027.3079x; 1.3601x over previous
import jax
import jax.numpy as jnp
from jax import lax
from jax.experimental import pallas as pl
from jax.experimental.pallas import tpu as pltpu
from jax.experimental.pallas import tpu_sc as plsc

S = 3000
SP = 3072
D_BASE = 8
D_EFF = 4
D_TAB = D_BASE + D_EFF
N = 1048576

NC = 2
NS = 16
NW = NC * NS
PER_W = N // NW
CHUNK = 2048
GROUPS = CHUNK // 16


def _table_tc_body(xt_ref, w1t_ref, b1_ref, w2t_ref, b2_ref, out_ref):
    h = jnp.dot(w1t_ref[...], xt_ref[...], preferred_element_type=jnp.float32)
    h = h + b1_ref[...]
    h = h * (1.0 / (1.0 + jnp.exp(-h)))
    t = jnp.dot(w2t_ref[...], h, preferred_element_type=jnp.float32)
    t = t + b2_ref[...]
    r = lax.broadcasted_iota(jnp.int32, (16, SP), 0)
    e = jnp.exp(t)
    t = jnp.where((r == 0) | (r == 8), e, jnp.where(r == 1, -e, t))
    out_ref[...] = t


def _make_table(spatial_features, Wb1, bb1, Wb2, bb2, We1, be1, We2, be2):
    xt = jnp.zeros((32, SP), jnp.float32).at[:, :S].set(spatial_features.T)
    w1t = jnp.concatenate([Wb1.T, We1.T], axis=0)
    b1 = jnp.concatenate([bb1, be1])[:, None]
    w2t = jnp.zeros((16, 64), jnp.float32)
    w2t = w2t.at[:D_BASE, :32].set(Wb2.T).at[D_BASE:D_TAB, 32:].set(We2.T)
    b2 = jnp.zeros((16,), jnp.float32)
    b2 = b2.at[:D_BASE].set(bb2).at[D_BASE:D_TAB].set(be2)
    return pl.pallas_call(
        _table_tc_body,
        out_shape=jax.ShapeDtypeStruct((16, SP), jnp.float32),
    )(xt, w1t, b1, w2t, b2[:, None])


NCHUNKS = PER_W // CHUNK


def _sc_body(tt_hbm, loc_hbm, off_hbm, al_hbm, bft_hbm, eft_hbm,
             eff_hbm, base_hbm, outc_hbm,
             tab_v, loc_v, off_v, al_v, bf_v, ef_v, eff_o, base_o, outc_o,
             in_sem, out_sem):
    wid = lax.axis_index("c") * NS + lax.axis_index("s")
    w0 = wid * PER_W
    pltpu.sync_copy(tt_hbm, tab_v)

    lanes = lax.broadcasted_iota(jnp.int32, (16,), 0)

    def start_in(ci, b):
        row0 = w0 + ci * CHUNK
        return [
            pltpu.async_copy(loc_hbm.at[pl.ds(row0, CHUNK)], loc_v.at[b], in_sem.at[b]),
            pltpu.async_copy(off_hbm.at[pl.ds(row0, CHUNK)], off_v.at[b], in_sem.at[b]),
            pltpu.async_copy(al_hbm.at[pl.ds(row0, CHUNK)], al_v.at[b], in_sem.at[b]),
            pltpu.async_copy(bft_hbm.at[:, pl.ds(row0, CHUNK)], bf_v.at[b], in_sem.at[b]),
            pltpu.async_copy(eft_hbm.at[:, pl.ds(row0, CHUNK)], ef_v.at[b], in_sem.at[b]),
        ]

    def start_out(ci, b):
        row0 = w0 + ci * CHUNK
        return [
            pltpu.async_copy(eff_o.at[b], eff_hbm.at[pl.ds(row0, CHUNK)], out_sem.at[b]),
            pltpu.async_copy(base_o.at[b], base_hbm.at[pl.ds(row0, CHUNK)], out_sem.at[b]),
            pltpu.async_copy(outc_o.at[b], outc_hbm.at[pl.ds(row0, CHUNK)], out_sem.at[b]),
        ]

    def compute(b):
        def group_body(g, _):
            base = g * 16
            idx = loc_v[b, pl.ds(base, 16)]
            acc_b = jnp.zeros((16,), jnp.float32)
            for j in range(D_BASE):
                cj = plsc.load_gather(tab_v, [idx + (j * SP)])
                fj = bf_v[b, j, pl.ds(base, 16)]
                acc_b = acc_b + cj * fj
            acc_e = jnp.zeros((16,), jnp.float32)
            for j in range(D_EFF):
                cj = plsc.load_gather(tab_v, [idx + ((D_BASE + j) * SP)])
                fj = ef_v[b, j, pl.ds(base, 16)]
                acc_e = acc_e + cj * fj
            baseline = jnp.exp(jnp.minimum(acc_b, 10.0))
            eff = 1.0 / (1.0 + jnp.exp(4.0 - acc_e))
            eff = jnp.clip(eff, 1e-6, 1.0 - 1e-6)
            ov = off_v[b, pl.ds(base, 16)]
            av = al_v[b, pl.ds(base, 16)]
            outcome = ov * baseline * (1.0 - av * eff)
            eff_o[b, pl.ds(base, 16)] = eff
            base_o[b, pl.ds(base, 16)] = baseline
            outc_o[b, pl.ds(base, 16)] = outcome
            return ()

        lax.fori_loop(0, GROUPS, group_body, (), unroll=2)

    pend_in = start_in(0, 0)
    pend_out = [None, None]
    for ci in range(NCHUNKS):
        b = ci % 2
        if ci + 1 < NCHUNKS:
            nxt = start_in(ci + 1, 1 - b)
        for h in pend_in:
            h.wait()
        if ci + 1 < NCHUNKS:
            pend_in = nxt
        if pend_out[b] is not None:
            for h in pend_out[b]:
                h.wait()
        compute(b)
        pend_out[b] = start_out(ci, b)
    for hs in pend_out:
        if hs is not None:
            for h in hs:
                h.wait()


def _sc_compute(tt_flat, loc_ind, offset, alert, bf_t, ef_t):
    mesh = plsc.VectorSubcoreMesh(core_axis_name="c", subcore_axis_name="s")
    f = pl.kernel(
        _sc_body,
        out_type=(jax.ShapeDtypeStruct((N,), jnp.float32),
                  jax.ShapeDtypeStruct((N,), jnp.float32),
                  jax.ShapeDtypeStruct((N,), jnp.float32)),
        mesh=mesh,
        compiler_params=pltpu.CompilerParams(
            needs_layout_passes=False, use_tc_tiling_on_sc=True),
        scratch_types=[
            pltpu.VMEM((16 * SP,), jnp.float32),
            pltpu.VMEM((2, CHUNK), jnp.int32),
            pltpu.VMEM((2, CHUNK), jnp.float32),
            pltpu.VMEM((2, CHUNK), jnp.float32),
            pltpu.VMEM((2, D_BASE, CHUNK), jnp.float32),
            pltpu.VMEM((2, D_EFF, CHUNK), jnp.float32),
            pltpu.VMEM((2, CHUNK), jnp.float32),
            pltpu.VMEM((2, CHUNK), jnp.float32),
            pltpu.VMEM((2, CHUNK), jnp.float32),
            pltpu.SemaphoreType.DMA((2,)),
            pltpu.SemaphoreType.DMA((2,)),
        ],
    )
    return f(tt_flat, loc_ind, offset, alert, bf_t, ef_t)


def kernel(hosps, loc_ind, offset, alert, baseline_features, eff_features, index,
           spatial_features, Wb1, bb1, Wb2, bb2, We1, be1, We2, be2):
    tt = _make_table(spatial_features, Wb1, bb1, Wb2, bb2, We1, be1, We2, be2)
    eff, baseline, outcome = _sc_compute(
        tt.reshape(-1), loc_ind, offset, alert,
        baseline_features.T, eff_features.T)
    return jnp.stack([eff, baseline, outcome], axis=1)


# speedup vs baseline: 1152.5876x; 1.1219x over previous
import jax
import jax.numpy as jnp
from jax import lax
from jax.experimental import pallas as pl
from jax.experimental.pallas import tpu as pltpu
from jax.experimental.pallas import tpu_sc as plsc

S = 3000
SP = 3072
D_BASE = 8
D_EFF = 4
D_TAB = D_BASE + D_EFF
N = 1048576

NC = 2
NS = 16
NW = NC * NS
PER_W = N // NW
CHUNK = 2048
GROUPS = CHUNK // 16


def _table_tc_body(xt_ref, w1t_ref, b1_ref, w2t_ref, b2_ref, out_ref):
    h = jnp.dot(w1t_ref[...], xt_ref[...], preferred_element_type=jnp.float32)
    h = h + b1_ref[...]
    h = h * (1.0 / (1.0 + jnp.exp(-h)))
    t = jnp.dot(w2t_ref[...], h, preferred_element_type=jnp.float32)
    t = t + b2_ref[...]
    r = lax.broadcasted_iota(jnp.int32, (16, SP), 0)
    e = jnp.exp(t)
    t = jnp.where((r == 0) | (r == 8), e, jnp.where(r == 1, -e, t))
    out_ref[...] = t


def _make_table(spatial_features, Wb1, bb1, Wb2, bb2, We1, be1, We2, be2):
    xt = jnp.zeros((32, SP), jnp.float32).at[:, :S].set(spatial_features.T)
    w1t = jnp.concatenate([Wb1.T, We1.T], axis=0)
    b1 = jnp.concatenate([bb1, be1])[:, None]
    w2t = jnp.zeros((16, 64), jnp.float32)
    w2t = w2t.at[:D_BASE, :32].set(Wb2.T).at[D_BASE:D_TAB, 32:].set(We2.T)
    b2 = jnp.zeros((16,), jnp.float32)
    b2 = b2.at[:D_BASE].set(bb2).at[D_BASE:D_TAB].set(be2)
    return pl.pallas_call(
        _table_tc_body,
        out_shape=jax.ShapeDtypeStruct((16, SP), jnp.float32),
    )(xt, w1t, b1, w2t, b2[:, None])


NCHUNKS = PER_W // CHUNK


def _sc_body(tt_hbm, loc_hbm, off_hbm, al_hbm, bft_hbm, eft_hbm,
             out_hbm,
             tab_v, loc_v, off_v, al_v, bf_v, ef_v, out_v,
             in_sem, out_sem):
    wid = lax.axis_index("c") * NS + lax.axis_index("s")
    w0 = wid * PER_W
    pltpu.sync_copy(tt_hbm, tab_v)

    lanes = lax.broadcasted_iota(jnp.int32, (16,), 0)

    def start_in(ci, b):
        row0 = w0 + ci * CHUNK
        return [
            pltpu.async_copy(loc_hbm.at[pl.ds(row0, CHUNK)], loc_v.at[b], in_sem.at[b]),
            pltpu.async_copy(off_hbm.at[pl.ds(row0, CHUNK)], off_v.at[b], in_sem.at[b]),
            pltpu.async_copy(al_hbm.at[pl.ds(row0, CHUNK)], al_v.at[b], in_sem.at[b]),
            pltpu.async_copy(bft_hbm.at[:, pl.ds(row0, CHUNK)], bf_v.at[b], in_sem.at[b]),
            pltpu.async_copy(eft_hbm.at[:, pl.ds(row0, CHUNK)], ef_v.at[b], in_sem.at[b]),
        ]

    def start_out(ci, b):
        row0 = w0 + ci * CHUNK
        return [
            pltpu.async_copy(out_v.at[b], out_hbm.at[pl.ds(row0 * 4, CHUNK * 4)], out_sem.at[b]),
        ]

    def compute(b):
        def group_body(g, _):
            base = g * 16
            idx = loc_v[b, pl.ds(base, 16)]
            acc_b = jnp.zeros((16,), jnp.float32)
            for j in range(D_BASE):
                cj = plsc.load_gather(tab_v, [idx + (j * SP)])
                fj = bf_v[b, j, pl.ds(base, 16)]
                acc_b = acc_b + cj * fj
            acc_e = jnp.zeros((16,), jnp.float32)
            for j in range(D_EFF):
                cj = plsc.load_gather(tab_v, [idx + ((D_BASE + j) * SP)])
                fj = ef_v[b, j, pl.ds(base, 16)]
                acc_e = acc_e + cj * fj
            baseline = jnp.exp(jnp.minimum(acc_b, 10.0))
            eff = 1.0 / (1.0 + jnp.exp(4.0 - acc_e))
            eff = jnp.clip(eff, 1e-6, 1.0 - 1e-6)
            ov = off_v[b, pl.ds(base, 16)]
            av = al_v[b, pl.ds(base, 16)]
            outcome = ov * baseline * (1.0 - av * eff)
            o = (g // 8) * 512 + (g % 8) * 16
            out_v[b, pl.ds(o, 16)] = eff
            out_v[b, pl.ds(o + 128, 16)] = baseline
            out_v[b, pl.ds(o + 256, 16)] = outcome
            return ()

        lax.fori_loop(0, GROUPS, group_body, (), unroll=2)

    pend_in = start_in(0, 0)
    pend_out = [None, None]
    for ci in range(NCHUNKS):
        b = ci % 2
        if ci + 1 < NCHUNKS:
            nxt = start_in(ci + 1, 1 - b)
        for h in pend_in:
            h.wait()
        if ci + 1 < NCHUNKS:
            pend_in = nxt
        if pend_out[b] is not None:
            for h in pend_out[b]:
                h.wait()
        compute(b)
        pend_out[b] = start_out(ci, b)
    for hs in pend_out:
        if hs is not None:
            for h in hs:
                h.wait()


def _sc_compute(tt_flat, loc_ind, offset, alert, bf_t, ef_t):
    mesh = plsc.VectorSubcoreMesh(core_axis_name="c", subcore_axis_name="s")
    f = pl.kernel(
        _sc_body,
        out_type=jax.ShapeDtypeStruct((N * 4,), jnp.float32),
        mesh=mesh,
        compiler_params=pltpu.CompilerParams(
            needs_layout_passes=False, use_tc_tiling_on_sc=True),
        scratch_types=[
            pltpu.VMEM((16 * SP,), jnp.float32),
            pltpu.VMEM((2, CHUNK), jnp.int32),
            pltpu.VMEM((2, CHUNK), jnp.float32),
            pltpu.VMEM((2, CHUNK), jnp.float32),
            pltpu.VMEM((2, D_BASE, CHUNK), jnp.float32),
            pltpu.VMEM((2, D_EFF, CHUNK), jnp.float32),
            pltpu.VMEM((2, CHUNK * 4), jnp.float32),
            pltpu.SemaphoreType.DMA((2,)),
            pltpu.SemaphoreType.DMA((2,)),
        ],
    )
    return f(tt_flat, loc_ind, offset, alert, bf_t, ef_t)


def kernel(hosps, loc_ind, offset, alert, baseline_features, eff_features, index,
           spatial_features, Wb1, bb1, Wb2, bb2, We1, be1, We2, be2):
    tt = _make_table(spatial_features, Wb1, bb1, Wb2, bb2, We1, be1, We2, be2)
    out4 = _sc_compute(
        tt.reshape(-1), loc_ind, offset, alert,
        baseline_features.T, eff_features.T)
    out = out4.reshape(N // 128, 4, 128).transpose(0, 2, 1).reshape(N, 4)
    return out[:, :3]


# speedup vs baseline: 1666.4637x; 1.4458x over previous
import jax
import jax.numpy as jnp
from jax import lax
from jax.experimental import pallas as pl
from jax.experimental.pallas import tpu as pltpu
from jax.experimental.pallas import tpu_sc as plsc

S = 3000
SP = 3072
D_BASE = 8
D_EFF = 4
D_TAB = D_BASE + D_EFF
N = 1048576

NC = 2
NS = 16
NW = NC * NS
PER_W = N // NW
CHUNK = 2048
GROUPS = CHUNK // 16


def _table_tc_body(xt_ref, w1t_ref, b1_ref, w2t_ref, b2_ref, out_ref):
    h = jnp.dot(w1t_ref[...], xt_ref[...], preferred_element_type=jnp.float32)
    h = h + b1_ref[...]
    h = h * (1.0 / (1.0 + jnp.exp(-h)))
    t = jnp.dot(w2t_ref[...], h, preferred_element_type=jnp.float32)
    t = t + b2_ref[...]
    r = lax.broadcasted_iota(jnp.int32, (16, SP), 0)
    e = jnp.exp(t)
    t = jnp.where((r == 0) | (r == 8), e, jnp.where(r == 1, -e, t))
    out_ref[...] = t


def _make_table(spatial_features, Wb1, bb1, Wb2, bb2, We1, be1, We2, be2):
    xt = jnp.zeros((32, SP), jnp.float32).at[:, :S].set(spatial_features.T)
    w1t = jnp.concatenate([Wb1.T, We1.T], axis=0)
    b1 = jnp.concatenate([bb1, be1])[:, None]
    w2t = jnp.zeros((16, 64), jnp.float32)
    w2t = w2t.at[:D_BASE, :32].set(Wb2.T).at[D_BASE:D_TAB, 32:].set(We2.T)
    b2 = jnp.zeros((16,), jnp.float32)
    b2 = b2.at[:D_BASE].set(bb2).at[D_BASE:D_TAB].set(be2)
    return pl.pallas_call(
        _table_tc_body,
        out_shape=jax.ShapeDtypeStruct((16, SP), jnp.float32),
    )(xt, w1t, b1, w2t, b2[:, None])


NCHUNKS = PER_W // CHUNK


def _sc_body(tt_hbm, loc_hbm, off_hbm, al_hbm, bft_hbm, eft_hbm,
             out_hbm,
             tab_v, loc_v, off_v, al_v, bf_v, ef_v, out_v,
             in_sem, out_sem):
    wid = lax.axis_index("c") * NS + lax.axis_index("s")
    w0 = wid * PER_W
    pltpu.sync_copy(tt_hbm, tab_v)

    lanes = lax.broadcasted_iota(jnp.int32, (16,), 0)

    def start_in(ci, b):
        row0 = w0 + ci * CHUNK
        return [
            pltpu.async_copy(loc_hbm.at[pl.ds(row0, CHUNK)], loc_v.at[b], in_sem.at[b]),
            pltpu.async_copy(off_hbm.at[pl.ds(row0, CHUNK)], off_v.at[b], in_sem.at[b]),
            pltpu.async_copy(al_hbm.at[pl.ds(row0, CHUNK)], al_v.at[b], in_sem.at[b]),
            pltpu.async_copy(bft_hbm.at[:, pl.ds(row0, CHUNK)], bf_v.at[b], in_sem.at[b]),
            pltpu.async_copy(eft_hbm.at[:, pl.ds(row0, CHUNK)], ef_v.at[b], in_sem.at[b]),
        ]

    def start_out(ci, b):
        row0 = w0 + ci * CHUNK
        return [
            pltpu.async_copy(out_v.at[b], out_hbm.at[pl.ds(row0 * 4, CHUNK * 4)], out_sem.at[b]),
        ]

    def compute(b):
        @plsc.parallel_loop(0, GROUPS, unroll=2)
        def group_body(g):
            base = g * 16
            idx = loc_v[b, pl.ds(base, 16)]
            acc_b = jnp.zeros((16,), jnp.float32)
            for j in range(D_BASE):
                cj = plsc.load_gather(tab_v, [idx + (j * SP)])
                fj = bf_v[b, j, pl.ds(base, 16)]
                acc_b = acc_b + cj * fj
            acc_e = jnp.zeros((16,), jnp.float32)
            for j in range(D_EFF):
                cj = plsc.load_gather(tab_v, [idx + ((D_BASE + j) * SP)])
                fj = ef_v[b, j, pl.ds(base, 16)]
                acc_e = acc_e + cj * fj
            baseline = jnp.exp(jnp.minimum(acc_b, 10.0))
            eff = 1.0 / (1.0 + jnp.exp(4.0 - acc_e))
            eff = jnp.clip(eff, 1e-6, 1.0 - 1e-6)
            ov = off_v[b, pl.ds(base, 16)]
            av = al_v[b, pl.ds(base, 16)]
            outcome = ov * baseline * (1.0 - av * eff)
            o = (g // 8) * 512 + (g % 8) * 16
            out_v[b, pl.ds(o, 16)] = eff
            out_v[b, pl.ds(o + 128, 16)] = baseline
            out_v[b, pl.ds(o + 256, 16)] = outcome

    pend_in = start_in(0, 0)
    pend_out = [None, None]
    for ci in range(NCHUNKS):
        b = ci % 2
        if ci + 1 < NCHUNKS:
            nxt = start_in(ci + 1, 1 - b)
        for h in pend_in:
            h.wait()
        if ci + 1 < NCHUNKS:
            pend_in = nxt
        if pend_out[b] is not None:
            for h in pend_out[b]:
                h.wait()
        compute(b)
        pend_out[b] = start_out(ci, b)
    for hs in pend_out:
        if hs is not None:
            for h in hs:
                h.wait()


def _sc_compute(tt_flat, loc_ind, offset, alert, bf_t, ef_t):
    mesh = plsc.VectorSubcoreMesh(core_axis_name="c", subcore_axis_name="s")
    f = pl.kernel(
        _sc_body,
        out_type=jax.ShapeDtypeStruct((N * 4,), jnp.float32),
        mesh=mesh,
        compiler_params=pltpu.CompilerParams(
            needs_layout_passes=False, use_tc_tiling_on_sc=True),
        scratch_types=[
            pltpu.VMEM((16 * SP,), jnp.float32),
            pltpu.VMEM((2, CHUNK), jnp.int32),
            pltpu.VMEM((2, CHUNK), jnp.float32),
            pltpu.VMEM((2, CHUNK), jnp.float32),
            pltpu.VMEM((2, D_BASE, CHUNK), jnp.float32),
            pltpu.VMEM((2, D_EFF, CHUNK), jnp.float32),
            pltpu.VMEM((2, CHUNK * 4), jnp.float32),
            pltpu.SemaphoreType.DMA((2,)),
            pltpu.SemaphoreType.DMA((2,)),
        ],
    )
    return f(tt_flat, loc_ind, offset, alert, bf_t, ef_t)


def kernel(hosps, loc_ind, offset, alert, baseline_features, eff_features, index,
           spatial_features, Wb1, bb1, Wb2, bb2, We1, be1, We2, be2):
    tt = _make_table(spatial_features, Wb1, bb1, Wb2, bb2, We1, be1, We2, be2)
    out4 = _sc_compute(
        tt.reshape(-1), loc_ind, offset, alert,
        baseline_features.T, eff_features.T)
    out = out4.reshape(N // 128, 4, 128).transpose(0, 2, 1).reshape(N, 4)
    return out[:, :3]


# speedup vs baseline: 1672.6549x; 1.0037x over previous
import jax
import jax.numpy as jnp
from jax import lax
from jax.experimental import pallas as pl
from jax.experimental.pallas import tpu as pltpu
from jax.experimental.pallas import tpu_sc as plsc

S = 3000
SP = 3072
D_BASE = 8
D_EFF = 4
D_TAB = D_BASE + D_EFF
N = 1048576

NC = 2
NS = 16
NW = NC * NS
PER_W = N // NW
CHUNK = 2048
GROUPS = CHUNK // 16


def _table_tc_body(xt_ref, w1t_ref, b1_ref, w2t_ref, b2_ref, out_ref):
    h = jnp.dot(w1t_ref[...], xt_ref[...], preferred_element_type=jnp.float32)
    h = h + b1_ref[...]
    h = h * (1.0 / (1.0 + jnp.exp(-h)))
    t = jnp.dot(w2t_ref[...], h, preferred_element_type=jnp.float32)
    t = t + b2_ref[...]
    r = lax.broadcasted_iota(jnp.int32, (16, SP), 0)
    e = jnp.exp(t)
    t = jnp.where((r == 0) | (r == 8), e, jnp.where(r == 1, -e, t))
    out_ref[...] = t


def _make_table(spatial_features, Wb1, bb1, Wb2, bb2, We1, be1, We2, be2):
    xt = jnp.zeros((32, SP), jnp.float32).at[:, :S].set(spatial_features.T)
    w1t = jnp.concatenate([Wb1.T, We1.T], axis=0)
    b1 = jnp.concatenate([bb1, be1])[:, None]
    w2t = jnp.zeros((16, 64), jnp.float32)
    w2t = w2t.at[:D_BASE, :32].set(Wb2.T).at[D_BASE:D_TAB, 32:].set(We2.T)
    b2 = jnp.zeros((16,), jnp.float32)
    b2 = b2.at[:D_BASE].set(bb2).at[D_BASE:D_TAB].set(be2)
    return pl.pallas_call(
        _table_tc_body,
        out_shape=jax.ShapeDtypeStruct((16, SP), jnp.float32),
    )(xt, w1t, b1, w2t, b2[:, None])


NCHUNKS = PER_W // CHUNK


def _sc_body(tt_hbm, loc_hbm, off_hbm, al_hbm, bft_hbm, eft_hbm,
             out_hbm,
             tab_v, loc_v, off_v, al_v, bf_v, ef_v, out_v,
             in_sem, out_sem):
    wid = lax.axis_index("c") * NS + lax.axis_index("s")
    w0 = wid * PER_W
    pltpu.sync_copy(tt_hbm, tab_v)

    lanes = lax.broadcasted_iota(jnp.int32, (16,), 0)

    def start_in(ci, b):
        row0 = w0 + ci * CHUNK
        return [
            pltpu.async_copy(loc_hbm.at[pl.ds(row0, CHUNK)], loc_v.at[b], in_sem.at[b]),
            pltpu.async_copy(off_hbm.at[pl.ds(row0, CHUNK)], off_v.at[b], in_sem.at[b]),
            pltpu.async_copy(al_hbm.at[pl.ds(row0, CHUNK)], al_v.at[b], in_sem.at[b]),
            pltpu.async_copy(bft_hbm.at[:, pl.ds(row0, CHUNK)], bf_v.at[b], in_sem.at[b]),
            pltpu.async_copy(eft_hbm.at[:, pl.ds(row0, CHUNK)], ef_v.at[b], in_sem.at[b]),
        ]

    def start_out(ci, b):
        row0 = w0 + ci * CHUNK
        return [
            pltpu.async_copy(out_v.at[b], out_hbm.at[pl.ds(row0 * 4, CHUNK * 4)], out_sem.at[b]),
        ]

    def compute(b):
        @plsc.parallel_loop(0, GROUPS, unroll=4)
        def group_body(g):
            base = g * 16
            idx = loc_v[b, pl.ds(base, 16)]
            acc_b = jnp.zeros((16,), jnp.float32)
            for j in range(D_BASE):
                cj = plsc.load_gather(tab_v, [idx + (j * SP)])
                fj = bf_v[b, j, pl.ds(base, 16)]
                acc_b = acc_b + cj * fj
            acc_e = jnp.zeros((16,), jnp.float32)
            for j in range(D_EFF):
                cj = plsc.load_gather(tab_v, [idx + ((D_BASE + j) * SP)])
                fj = ef_v[b, j, pl.ds(base, 16)]
                acc_e = acc_e + cj * fj
            baseline = jnp.exp(jnp.minimum(acc_b, 10.0))
            eff = 1.0 / (1.0 + jnp.exp(4.0 - acc_e))
            eff = jnp.clip(eff, 1e-6, 1.0 - 1e-6)
            ov = off_v[b, pl.ds(base, 16)]
            av = al_v[b, pl.ds(base, 16)]
            outcome = ov * baseline * (1.0 - av * eff)
            o = (g // 8) * 512 + (g % 8) * 16
            out_v[b, pl.ds(o, 16)] = eff
            out_v[b, pl.ds(o + 128, 16)] = baseline
            out_v[b, pl.ds(o + 256, 16)] = outcome

    pend_in = start_in(0, 0)
    pend_out = [None, None]
    for ci in range(NCHUNKS):
        b = ci % 2
        if ci + 1 < NCHUNKS:
            nxt = start_in(ci + 1, 1 - b)
        for h in pend_in:
            h.wait()
        if ci + 1 < NCHUNKS:
            pend_in = nxt
        if pend_out[b] is not None:
            for h in pend_out[b]:
                h.wait()
        compute(b)
        pend_out[b] = start_out(ci, b)
    for hs in pend_out:
        if hs is not None:
            for h in hs:
                h.wait()


def _sc_compute(tt_flat, loc_ind, offset, alert, bf_t, ef_t):
    mesh = plsc.VectorSubcoreMesh(core_axis_name="c", subcore_axis_name="s")
    f = pl.kernel(
        _sc_body,
        out_type=jax.ShapeDtypeStruct((N * 4,), jnp.float32),
        mesh=mesh,
        compiler_params=pltpu.CompilerParams(
            needs_layout_passes=False, use_tc_tiling_on_sc=True),
        scratch_types=[
            pltpu.VMEM((16 * SP,), jnp.float32),
            pltpu.VMEM((2, CHUNK), jnp.int32),
            pltpu.VMEM((2, CHUNK), jnp.float32),
            pltpu.VMEM((2, CHUNK), jnp.float32),
            pltpu.VMEM((2, D_BASE, CHUNK), jnp.float32),
            pltpu.VMEM((2, D_EFF, CHUNK), jnp.float32),
            pltpu.VMEM((2, CHUNK * 4), jnp.float32),
            pltpu.SemaphoreType.DMA((2,)),
            pltpu.SemaphoreType.DMA((2,)),
        ],
    )
    return f(tt_flat, loc_ind, offset, alert, bf_t, ef_t)


def kernel(hosps, loc_ind, offset, alert, baseline_features, eff_features, index,
           spatial_features, Wb1, bb1, Wb2, bb2, We1, be1, We2, be2):
    tt = _make_table(spatial_features, Wb1, bb1, Wb2, bb2, We1, be1, We2, be2)
    out4 = _sc_compute(
        tt.reshape(-1), loc_ind, offset, alert,
        baseline_features.T, eff_features.T)
    out = out4.reshape(N // 128, 4, 128).transpose(0, 2, 1).reshape(N, 4)
    return out[:, :3]


# speedup vs baseline: 1840.8015x; 1.1005x over previous
import jax
import jax.numpy as jnp
from jax import lax
from jax.experimental import pallas as pl
from jax.experimental.pallas import tpu as pltpu
from jax.experimental.pallas import tpu_sc as plsc

S = 3000
SP = 3072
D_BASE = 8
D_EFF = 4
D_TAB = D_BASE + D_EFF
N = 1048576

NC = 2
NS = 16
NW = NC * NS
PER_W = N // NW
CHUNK = 2048
GROUPS = CHUNK // 16


def _table_tc_body(xt_ref, w1_ref, b1_ref, w2_ref, b2_ref, out_ref):
    h = lax.dot_general(w1_ref[...], xt_ref[...], (((0,), (0,)), ((), ())),
                        preferred_element_type=jnp.float32)
    h = h + b1_ref[...]
    h = h * (1.0 / (1.0 + jnp.exp(-h)))
    t = lax.dot_general(w2_ref[...], h, (((0,), (0,)), ((), ())),
                        preferred_element_type=jnp.float32)
    t = t + b2_ref[...]
    r = lax.broadcasted_iota(jnp.int32, (16, S), 0)
    e = jnp.exp(t)
    t = jnp.where((r == 0) | (r == 8), e, jnp.where(r == 1, -e, t))
    out_ref[...] = t


def _make_table(spatial_features, Wb1, bb1, Wb2, bb2, We1, be1, We2, be2):
    w1 = jnp.concatenate([Wb1, We1], axis=1)
    b1 = jnp.concatenate([bb1, be1]).reshape(64, 1)
    w2 = jnp.zeros((64, 16), jnp.float32)
    w2 = w2.at[:32, :D_BASE].set(Wb2).at[32:, D_BASE:D_TAB].set(We2)
    b2 = jnp.zeros((16,), jnp.float32)
    b2 = b2.at[:D_BASE].set(bb2).at[D_BASE:D_TAB].set(be2)
    return pl.pallas_call(
        _table_tc_body,
        out_shape=jax.ShapeDtypeStruct((16, S), jnp.float32),
    )(spatial_features.T, w1, b1, w2, b2.reshape(16, 1))


NCHUNKS = PER_W // CHUNK


def _sc_body(tt_hbm, loc_hbm, off_hbm, al_hbm, bft_hbm, eft_hbm,
             out_hbm,
             tab_v, loc_v, off_v, al_v, bf_v, ef_v, out_v,
             in_sem, out_sem):
    wid = lax.axis_index("c") * NS + lax.axis_index("s")
    w0 = wid * PER_W
    pltpu.sync_copy(tt_hbm, tab_v)

    lanes = lax.broadcasted_iota(jnp.int32, (16,), 0)

    def start_in(ci, b):
        row0 = w0 + ci * CHUNK
        return [
            pltpu.async_copy(loc_hbm.at[pl.ds(row0, CHUNK)], loc_v.at[b], in_sem.at[b]),
            pltpu.async_copy(off_hbm.at[pl.ds(row0, CHUNK)], off_v.at[b], in_sem.at[b]),
            pltpu.async_copy(al_hbm.at[pl.ds(row0, CHUNK)], al_v.at[b], in_sem.at[b]),
            pltpu.async_copy(bft_hbm.at[:, pl.ds(row0, CHUNK)], bf_v.at[b], in_sem.at[b]),
            pltpu.async_copy(eft_hbm.at[:, pl.ds(row0, CHUNK)], ef_v.at[b], in_sem.at[b]),
        ]

    def start_out(ci, b):
        row0 = w0 + ci * CHUNK
        return [
            pltpu.async_copy(out_v.at[b], out_hbm.at[pl.ds(row0 * 4, CHUNK * 4)], out_sem.at[b]),
        ]

    def compute(b):
        @plsc.parallel_loop(0, GROUPS, unroll=4)
        def group_body(g):
            base = g * 16
            idx = loc_v[b, pl.ds(base, 16)]
            acc_b = jnp.zeros((16,), jnp.float32)
            for j in range(D_BASE):
                cj = plsc.load_gather(tab_v, [idx + (j * S)])
                fj = bf_v[b, j, pl.ds(base, 16)]
                acc_b = acc_b + cj * fj
            acc_e = jnp.zeros((16,), jnp.float32)
            for j in range(D_EFF):
                cj = plsc.load_gather(tab_v, [idx + ((D_BASE + j) * S)])
                fj = ef_v[b, j, pl.ds(base, 16)]
                acc_e = acc_e + cj * fj
            baseline = jnp.exp(jnp.minimum(acc_b, 10.0))
            eff = 1.0 / (1.0 + jnp.exp(4.0 - acc_e))
            eff = jnp.clip(eff, 1e-6, 1.0 - 1e-6)
            ov = off_v[b, pl.ds(base, 16)]
            av = al_v[b, pl.ds(base, 16)]
            outcome = ov * baseline * (1.0 - av * eff)
            o = (g // 8) * 512 + (g % 8) * 16
            out_v[b, pl.ds(o, 16)] = eff
            out_v[b, pl.ds(o + 128, 16)] = baseline
            out_v[b, pl.ds(o + 256, 16)] = outcome

    pend_in = start_in(0, 0)
    pend_out = [None, None]
    for ci in range(NCHUNKS):
        b = ci % 2
        if ci + 1 < NCHUNKS:
            nxt = start_in(ci + 1, 1 - b)
        for h in pend_in:
            h.wait()
        if ci + 1 < NCHUNKS:
            pend_in = nxt
        if pend_out[b] is not None:
            for h in pend_out[b]:
                h.wait()
        compute(b)
        pend_out[b] = start_out(ci, b)
    for hs in pend_out:
        if hs is not None:
            for h in hs:
                h.wait()


def _sc_compute(tt_flat, loc_ind, offset, alert, bf_t, ef_t):
    mesh = plsc.VectorSubcoreMesh(core_axis_name="c", subcore_axis_name="s")
    f = pl.kernel(
        _sc_body,
        out_type=jax.ShapeDtypeStruct((N * 4,), jnp.float32),
        mesh=mesh,
        compiler_params=pltpu.CompilerParams(
            needs_layout_passes=False, use_tc_tiling_on_sc=True),
        scratch_types=[
            pltpu.VMEM((D_TAB * S,), jnp.float32),
            pltpu.VMEM((2, CHUNK), jnp.int32),
            pltpu.VMEM((2, CHUNK), jnp.float32),
            pltpu.VMEM((2, CHUNK), jnp.float32),
            pltpu.VMEM((2, D_BASE, CHUNK), jnp.float32),
            pltpu.VMEM((2, D_EFF, CHUNK), jnp.float32),
            pltpu.VMEM((2, CHUNK * 4), jnp.float32),
            pltpu.SemaphoreType.DMA((2,)),
            pltpu.SemaphoreType.DMA((2,)),
        ],
    )
    return f(tt_flat, loc_ind, offset, alert, bf_t, ef_t)


def kernel(hosps, loc_ind, offset, alert, baseline_features, eff_features, index,
           spatial_features, Wb1, bb1, Wb2, bb2, We1, be1, We2, be2):
    tt = _make_table(spatial_features, Wb1, bb1, Wb2, bb2, We1, be1, We2, be2)
    out4 = _sc_compute(
        tt[:D_TAB].reshape(-1), loc_ind, offset, alert,
        baseline_features.T, eff_features.T)
    out = out4.reshape(N // 128, 4, 128).transpose(0, 2, 1).reshape(N, 4)
    return out[:, :3]


# speedup vs baseline: 1868.1387x; 1.0149x over previous
import jax
import jax.numpy as jnp
from jax import lax
from jax.experimental import pallas as pl
from jax.experimental.pallas import tpu as pltpu
from jax.experimental.pallas import tpu_sc as plsc

S = 3000
SP = 3072
D_BASE = 8
D_EFF = 4
D_TAB = D_BASE + D_EFF
N = 1048576

NC = 2
NS = 16
NW = NC * NS
PER_W = N // NW
CHUNK = 2048
GROUPS = CHUNK // 16


def _table_tc_body(xt_ref, w1_ref, b1_ref, w2_ref, b2_ref, out_ref):
    h = lax.dot_general(w1_ref[...], xt_ref[...], (((0,), (0,)), ((), ())),
                        preferred_element_type=jnp.float32)
    h = h + b1_ref[...]
    h = h * (1.0 / (1.0 + jnp.exp(-h)))
    t = lax.dot_general(w2_ref[...], h, (((0,), (0,)), ((), ())),
                        preferred_element_type=jnp.float32)
    t = t + b2_ref[...]
    r = lax.broadcasted_iota(jnp.int32, (16, S), 0)
    e = jnp.exp(t)
    t = jnp.where((r == 0) | (r == 8), e, jnp.where(r == 1, -e, t))
    out_ref[...] = t


def _make_table(spatial_features, Wb1, bb1, Wb2, bb2, We1, be1, We2, be2):
    w1 = jnp.concatenate([Wb1, We1], axis=1)
    b1 = jnp.concatenate([bb1, be1]).reshape(64, 1)
    w2 = jnp.zeros((64, 16), jnp.float32)
    w2 = w2.at[:32, :D_BASE].set(Wb2).at[32:, D_BASE:D_TAB].set(We2)
    b2 = jnp.zeros((16,), jnp.float32)
    b2 = b2.at[:D_BASE].set(bb2).at[D_BASE:D_TAB].set(be2)
    return pl.pallas_call(
        _table_tc_body,
        out_shape=jax.ShapeDtypeStruct((16, S), jnp.float32),
    )(spatial_features.T, w1, b1, w2, b2.reshape(16, 1))


NCHUNKS = PER_W // CHUNK


def _sc_body(tt_hbm, loc_hbm, off_hbm, al_hbm, bft_hbm, eft_hbm,
             out_hbm,
             tab_v, loc_v, off_v, al_v, bf_v, ef_v, out_v,
             in_sem, out_sem):
    wid = lax.axis_index("c") * NS + lax.axis_index("s")
    w0 = wid * PER_W
    lanes = lax.broadcasted_iota(jnp.int32, (16,), 0)

    def start_in(ci, b):
        row0 = w0 + ci * CHUNK
        return [
            pltpu.async_copy(loc_hbm.at[pl.ds(row0, CHUNK)], loc_v.at[b], in_sem.at[b]),
            pltpu.async_copy(off_hbm.at[pl.ds(row0, CHUNK)], off_v.at[b], in_sem.at[b]),
            pltpu.async_copy(al_hbm.at[pl.ds(row0, CHUNK)], al_v.at[b], in_sem.at[b]),
            pltpu.async_copy(bft_hbm.at[:, pl.ds(row0, CHUNK)], bf_v.at[b], in_sem.at[b]),
            pltpu.async_copy(eft_hbm.at[:, pl.ds(row0, CHUNK)], ef_v.at[b], in_sem.at[b]),
        ]

    def start_out(ci, b):
        row0 = w0 + ci * CHUNK
        return [
            pltpu.async_copy(out_v.at[b], out_hbm.at[pl.ds(row0 * 4, CHUNK * 4)], out_sem.at[b]),
        ]

    def compute(b):
        @plsc.parallel_loop(0, GROUPS, unroll=4)
        def group_body(g):
            base = g * 16
            idx = loc_v[b, pl.ds(base, 16)]
            acc_b = jnp.zeros((16,), jnp.float32)
            for j in range(D_BASE):
                cj = plsc.load_gather(tab_v, [idx + (j * S)])
                fj = bf_v[b, j, pl.ds(base, 16)]
                acc_b = acc_b + cj * fj
            acc_e = jnp.zeros((16,), jnp.float32)
            for j in range(D_EFF):
                cj = plsc.load_gather(tab_v, [idx + ((D_BASE + j) * S)])
                fj = ef_v[b, j, pl.ds(base, 16)]
                acc_e = acc_e + cj * fj
            baseline = jnp.exp(jnp.minimum(acc_b, 10.0))
            eff = 1.0 / (1.0 + jnp.exp(4.0 - acc_e))
            eff = jnp.clip(eff, 1e-6, 1.0 - 1e-6)
            ov = off_v[b, pl.ds(base, 16)]
            av = al_v[b, pl.ds(base, 16)]
            outcome = ov * baseline * (1.0 - av * eff)
            o = (g // 8) * 512 + (g % 8) * 16
            out_v[b, pl.ds(o, 16)] = eff
            out_v[b, pl.ds(o + 128, 16)] = baseline
            out_v[b, pl.ds(o + 256, 16)] = outcome

    pend_in = start_in(0, 0)
    pltpu.sync_copy(tt_hbm, tab_v)
    pend_out = [None, None]
    for ci in range(NCHUNKS):
        b = ci % 2
        if ci + 1 < NCHUNKS:
            nxt = start_in(ci + 1, 1 - b)
        for h in pend_in:
            h.wait()
        if ci + 1 < NCHUNKS:
            pend_in = nxt
        if pend_out[b] is not None:
            for h in pend_out[b]:
                h.wait()
        compute(b)
        pend_out[b] = start_out(ci, b)
    for hs in pend_out:
        if hs is not None:
            for h in hs:
                h.wait()


def _sc_compute(tt_flat, loc_ind, offset, alert, bf_t, ef_t):
    mesh = plsc.VectorSubcoreMesh(core_axis_name="c", subcore_axis_name="s")
    f = pl.kernel(
        _sc_body,
        out_type=jax.ShapeDtypeStruct((N * 4,), jnp.float32),
        mesh=mesh,
        compiler_params=pltpu.CompilerParams(
            needs_layout_passes=False, use_tc_tiling_on_sc=True),
        scratch_types=[
            pltpu.VMEM((D_TAB * S,), jnp.float32),
            pltpu.VMEM((2, CHUNK), jnp.int32),
            pltpu.VMEM((2, CHUNK), jnp.float32),
            pltpu.VMEM((2, CHUNK), jnp.float32),
            pltpu.VMEM((2, D_BASE, CHUNK), jnp.float32),
            pltpu.VMEM((2, D_EFF, CHUNK), jnp.float32),
            pltpu.VMEM((2, CHUNK * 4), jnp.float32),
            pltpu.SemaphoreType.DMA((2,)),
            pltpu.SemaphoreType.DMA((2,)),
        ],
    )
    return f(tt_flat, loc_ind, offset, alert, bf_t, ef_t)


def kernel(hosps, loc_ind, offset, alert, baseline_features, eff_features, index,
           spatial_features, Wb1, bb1, Wb2, bb2, We1, be1, We2, be2):
    tt = _make_table(spatial_features, Wb1, bb1, Wb2, bb2, We1, be1, We2, be2)
    out4 = _sc_compute(
        tt[:D_TAB].reshape(-1), loc_ind, offset, alert,
        baseline_features.T, eff_features.T)
    out = out4.reshape(N // 128, 4, 128).transpose(0, 2, 1).reshape(N, 4)
    return out[:, :3]
